# baseline (speedup 1.0000x reference)
import numpy as np
import jax
import jax.numpy as jnp
from jax import lax
from jax.experimental import pallas as pl
from jax.experimental.pallas import tpu as pltpu

_H = 32
_W = 32
_CIN = 128
_COUT = 256
_KH = 3
_KW = 3
_HW = _H * _W
_K = _KH * _KW * _CIN
_B = 8


def _conv_body(x_ref, w_ref, o_ref, xc_ref):
    w_idx = lax.rem(lax.broadcasted_iota(jnp.int32, (_HW, 1), 0), _W)

    def build(b):
        base = b * _HW
        xb = x_ref[b]
        xl = jnp.where(w_idx == _W - 1, jnp.bfloat16(0), xb)
        xr = jnp.where(w_idx == 0, jnp.bfloat16(0), xb)
        taps = (xl, xb, xr)
        zeros = jnp.zeros((48, _K), jnp.bfloat16)
        xc_ref[pl.ds(base, 48), :] = zeros
        xc_ref[pl.ds(base + _HW - 48, 48), :] = zeros
        for dx in range(_KW):
            off = dx - 1
            lo = max(0, -off)
            hi = min(_HW, _HW - off)
            k1 = (_KW + dx) * _CIN
            xc_ref[pl.ds(base + lo, hi - lo), k1:k1 + _CIN] = (
                taps[dx][lo + off:hi + off])
        for dy in (0, 2):
            for dx in range(_KW):
                off = (dy - 1) * _W + (dx - 1)
                lo = max(0, -off)
                hi = min(_HW, _HW - off)
                k0 = (dy * _KW + dx) * _CIN
                k1 = (_KW + dx) * _CIN
                lo2 = max(lo, -(dy - 1) * _W)
                hi2 = min(hi, _HW - (dy - 1) * _W)
                xc_ref[pl.ds(base + lo2, hi2 - lo2), k0:k0 + _CIN] = (
                    xc_ref[pl.ds(base + lo2 + (dy - 1) * _W, hi2 - lo2),
                           k1:k1 + _CIN])
                if lo2 > lo:
                    xc_ref[pl.ds(base + lo, lo2 - lo), k0:k0 + _CIN] = (
                        taps[dx][lo + off:lo2 + off])
                if hi2 < hi:
                    xc_ref[pl.ds(base + hi2, hi - hi2), k0:k0 + _CIN] = (
                        taps[dx][hi2 + off:hi + off])

    def matmul(b):
        p = jnp.dot(xc_ref[pl.ds(b * _HW, _HW), :], w_ref[...],
                    preferred_element_type=jnp.float32)
        o_ref[b] = p.reshape(_HW, _COUT)

    build(0)
    for b in range(1, _B):
        build(b)
        matmul(b - 1)
    matmul(_B - 1)


def kernel(x, weight):
    n = x.shape[0]
    scale = 1.0 / float(np.sqrt(np.prod(weight.shape[1:])))
    w_t = jnp.transpose(weight, (2, 3, 1, 0)).reshape(_K, _COUT)
    w_t = (w_t * scale).astype(jnp.bfloat16)
    x_nhwc = jnp.transpose(x, (0, 2, 3, 1)).reshape(n, _HW, _CIN)
    x_nhwc = x_nhwc.astype(jnp.bfloat16)

    out = pl.pallas_call(
        _conv_body,
        out_shape=jax.ShapeDtypeStruct((n, _HW, _COUT), jnp.float32),
        grid=(n // _B,),
        in_specs=[
            pl.BlockSpec((_B, _HW, _CIN), lambda i: (i, 0, 0)),
            pl.BlockSpec((_K, _COUT), lambda i: (0, 0)),
        ],
        out_specs=pl.BlockSpec((_B, _HW, _COUT), lambda i: (i, 0, 0)),
        scratch_shapes=[pltpu.VMEM((_B * _HW, _K), jnp.bfloat16)],
        compiler_params=pltpu.CompilerParams(
            dimension_semantics=("parallel",),
            vmem_limit_bytes=64 * 1024 * 1024),
    )(x_nhwc, w_t)
    out = out.reshape(n, _H, _W, _COUT)
    return jnp.transpose(out, (0, 3, 1, 2))


# speedup vs baseline: 1.0513x; 1.0513x over previous
import numpy as np
import jax
import jax.numpy as jnp
from jax import lax
from jax.experimental import pallas as pl
from jax.experimental.pallas import tpu as pltpu

_H = 32
_W = 32
_CIN = 128
_COUT = 256
_KH = 3
_KW = 3
_HW = _H * _W
_K = _KH * _KW * _CIN
_B = 4


def _conv_body(x_ref, w_ref, o_ref, xc_ref):
    w_idx = lax.rem(lax.broadcasted_iota(jnp.int32, (_HW, 1), 0), _W)

    def build(b):
        base = b * _HW
        xb = x_ref[b]
        xl = jnp.where(w_idx == _W - 1, jnp.bfloat16(0), xb)
        xr = jnp.where(w_idx == 0, jnp.bfloat16(0), xb)
        taps = (xl, xb, xr)
        zeros = jnp.zeros((48, _K), jnp.bfloat16)
        xc_ref[pl.ds(base, 48), :] = zeros
        xc_ref[pl.ds(base + _HW - 48, 48), :] = zeros
        for dy in range(_KH):
            for dx in range(_KW):
                off = (dy - 1) * _W + (dx - 1)
                lo = max(0, -off)
                hi = min(_HW, _HW - off)
                k0 = (dy * _KW + dx) * _CIN
                xc_ref[pl.ds(base + lo, hi - lo), k0:k0 + _CIN] = (
                    taps[dx][lo + off:hi + off])

    def matmul(b):
        p = jnp.dot(xc_ref[pl.ds(b * _HW, _HW), :], w_ref[...],
                    preferred_element_type=jnp.float32)
        o_ref[b] = p.reshape(_HW, _COUT)

    build(0)
    for b in range(1, _B):
        build(b)
        matmul(b - 1)
    matmul(_B - 1)


def kernel(x, weight):
    n = x.shape[0]
    scale = 1.0 / float(np.sqrt(np.prod(weight.shape[1:])))
    w_t = jnp.transpose(weight, (2, 3, 1, 0)).reshape(_K, _COUT)
    w_t = (w_t * scale).astype(jnp.bfloat16)
    x_nhwc = jnp.transpose(x, (0, 2, 3, 1)).reshape(n, _HW, _CIN)
    x_nhwc = x_nhwc.astype(jnp.bfloat16)

    out = pl.pallas_call(
        _conv_body,
        out_shape=jax.ShapeDtypeStruct((n, _HW, _COUT), jnp.float32),
        grid=(n // _B,),
        in_specs=[
            pl.BlockSpec((_B, _HW, _CIN), lambda i: (i, 0, 0)),
            pl.BlockSpec((_K, _COUT), lambda i: (0, 0)),
        ],
        out_specs=pl.BlockSpec((_B, _HW, _COUT), lambda i: (i, 0, 0)),
        scratch_shapes=[pltpu.VMEM((_B * _HW, _K), jnp.bfloat16)],
        compiler_params=pltpu.CompilerParams(
            dimension_semantics=("parallel",),
            vmem_limit_bytes=64 * 1024 * 1024),
    )(x_nhwc, w_t)
    out = out.reshape(n, _H, _W, _COUT)
    return jnp.transpose(out, (0, 3, 1, 2))


# speedup vs baseline: 1.0523x; 1.0009x over previous
import numpy as np
import jax
import jax.numpy as jnp
from jax import lax
from jax.experimental import pallas as pl
from jax.experimental.pallas import tpu as pltpu

_H = 32
_W = 32
_CIN = 128
_COUT = 256
_KH = 3
_KW = 3
_HW = _H * _W
_K = _KH * _KW * _CIN
_B = 8


def _conv_body(x_ref, w_ref, o_ref, xc_ref):
    w_idx = lax.rem(lax.broadcasted_iota(jnp.int32, (_HW, 1), 0), _W)

    def build(b):
        base = b * _HW
        xb = x_ref[b]
        xl = jnp.where(w_idx == _W - 1, jnp.bfloat16(0), xb)
        xr = jnp.where(w_idx == 0, jnp.bfloat16(0), xb)
        taps = (xl, xb, xr)
        zeros = jnp.zeros((48, _K), jnp.bfloat16)
        xc_ref[pl.ds(base, 48), :] = zeros
        xc_ref[pl.ds(base + _HW - 48, 48), :] = zeros
        for dy in range(_KH):
            for dx in range(_KW):
                off = (dy - 1) * _W + (dx - 1)
                lo = max(0, -off)
                hi = min(_HW, _HW - off)
                k0 = (dy * _KW + dx) * _CIN
                xc_ref[pl.ds(base + lo, hi - lo), k0:k0 + _CIN] = (
                    taps[dx][lo + off:hi + off])

    def matmul(b):
        p = jnp.dot(xc_ref[pl.ds(b * _HW, _HW), :], w_ref[...],
                    preferred_element_type=jnp.float32)
        o_ref[b] = p.reshape(_HW, _COUT)

    build(0)
    for b in range(1, _B):
        build(b)
        matmul(b - 1)
    matmul(_B - 1)


def kernel(x, weight):
    n = x.shape[0]
    scale = 1.0 / float(np.sqrt(np.prod(weight.shape[1:])))
    w_t = jnp.transpose(weight, (2, 3, 1, 0)).reshape(_K, _COUT)
    w_t = (w_t * scale).astype(jnp.bfloat16)
    x_nhwc = jnp.transpose(x, (0, 2, 3, 1)).reshape(n, _HW, _CIN)
    x_nhwc = x_nhwc.astype(jnp.bfloat16)

    out = pl.pallas_call(
        _conv_body,
        out_shape=jax.ShapeDtypeStruct((n, _HW, _COUT), jnp.float32),
        grid=(n // _B,),
        in_specs=[
            pl.BlockSpec((_B, _HW, _CIN), lambda i: (i, 0, 0)),
            pl.BlockSpec((_K, _COUT), lambda i: (0, 0)),
        ],
        out_specs=pl.BlockSpec((_B, _HW, _COUT), lambda i: (i, 0, 0)),
        scratch_shapes=[pltpu.VMEM((_B * _HW, _K), jnp.bfloat16)],
        compiler_params=pltpu.CompilerParams(
            dimension_semantics=("parallel",),
            vmem_limit_bytes=64 * 1024 * 1024),
    )(x_nhwc, w_t)
    out = out.reshape(n, _H, _W, _COUT)
    return jnp.transpose(out, (0, 3, 1, 2))


# speedup vs baseline: 1.0548x; 1.0024x over previous
import numpy as np
import jax
import jax.numpy as jnp
from jax import lax
from jax.experimental import pallas as pl
from jax.experimental.pallas import tpu as pltpu

_H = 32
_W = 32
_CIN = 128
_COUT = 256
_KH = 3
_KW = 3
_HW = _H * _W
_K = _KH * _KW * _CIN
_B = 8


def _conv_body(x_ref, w_ref, o_ref, xc_ref):
    w_idx = lax.rem(lax.broadcasted_iota(jnp.int32, (_HW, 1), 0), _W)

    def build(b):
        base = b * _HW
        xb = x_ref[b]
        xl = jnp.where(w_idx == _W - 1, jnp.bfloat16(0), xb)
        xr = jnp.where(w_idx == 0, jnp.bfloat16(0), xb)
        taps = (xl, xb, xr)
        zeros = jnp.zeros((48, _K), jnp.bfloat16)
        xc_ref[pl.ds(base, 48), :] = zeros
        xc_ref[pl.ds(base + _HW - 48, 48), :] = zeros
        for dy in range(_KH):
            for dx in range(_KW):
                off = (dy - 1) * _W + (dx - 1)
                lo = max(0, -off)
                hi = min(_HW, _HW - off)
                k0 = (dy * _KW + dx) * _CIN
                xc_ref[pl.ds(base + lo, hi - lo), k0:k0 + _CIN] = (
                    taps[dx][lo + off:hi + off])

    def matmul(b):
        for h in range(2):
            p = jnp.dot(xc_ref[pl.ds(b * _HW + h * (_HW // 2), _HW // 2), :],
                        w_ref[...], preferred_element_type=jnp.float32)
            o_ref[b, pl.ds(h * (_HW // 2), _HW // 2)] = p

    build(0)
    for b in range(1, _B):
        build(b)
        matmul(b - 1)
    matmul(_B - 1)


def kernel(x, weight):
    n = x.shape[0]
    scale = 1.0 / float(np.sqrt(np.prod(weight.shape[1:])))
    w_t = jnp.transpose(weight, (2, 3, 1, 0)).reshape(_K, _COUT)
    w_t = (w_t * scale).astype(jnp.bfloat16)
    x_nhwc = jnp.transpose(x, (0, 2, 3, 1)).reshape(n, _HW, _CIN)
    x_nhwc = x_nhwc.astype(jnp.bfloat16)

    out = pl.pallas_call(
        _conv_body,
        out_shape=jax.ShapeDtypeStruct((n, _HW, _COUT), jnp.float32),
        grid=(n // _B,),
        in_specs=[
            pl.BlockSpec((_B, _HW, _CIN), lambda i: (i, 0, 0)),
            pl.BlockSpec((_K, _COUT), lambda i: (0, 0)),
        ],
        out_specs=pl.BlockSpec((_B, _HW, _COUT), lambda i: (i, 0, 0)),
        scratch_shapes=[pltpu.VMEM((_B * _HW, _K), jnp.bfloat16)],
        compiler_params=pltpu.CompilerParams(
            dimension_semantics=("parallel",),
            vmem_limit_bytes=64 * 1024 * 1024),
    )(x_nhwc, w_t)
    out = out.reshape(n, _H, _W, _COUT)
    return jnp.transpose(out, (0, 3, 1, 2))
